# Initial kernel scaffold; baseline (speedup 1.0000x reference)
import jax
import jax.numpy as jnp
from jax.experimental import pallas as pl


def kernel(x, edge_index, params):
    raise NotImplementedError("write your pallas kernel here")


# speedup vs baseline: 5.8209x; 5.8209x over previous
import functools

import jax
import jax.numpy as jnp
import numpy as np
from jax import lax
from jax.experimental import pallas as pl
from jax.experimental.pallas import tpu as pltpu
from jax.experimental.pallas import tpu_sc as plsc

N = 10000
NP = 10240
D = 128
R = 1024
CH = 64
AGGR = 10112
NSUB = 16
TRASH = N

_ILV = np.empty((D,), np.int32)
for _u in range(D // 32):
    for _i in range(16):
        _ILV[32 * _u + 2 * _i] = 32 * _u + _i
        _ILV[32 * _u + 2 * _i + 1] = 32 * _u + 16 + _i


def _dotT(a, w):
    return lax.dot_general(a, w, (((1,), (1,)), ((), ())),
                           preferred_element_type=jnp.float32)


def _emit_layer(h2, wd, bd, b2, k2_ref, qv2_ref, s2_ref):
    for d in range(2):
        h = h2[d]
        k = _dotT(h, wd[d, 0]) + bd[d, 0]
        q = _dotT(h, wd[d, 1]) + bd[d, 1]
        v = _dotT(h, wd[d, 2]) + bd[d, 2]
        s = _dotT(h, wd[d, 3]) + b2[d]
        k2_ref[d] = k
        qv2_ref[d, :, 0:D] = q.astype(jnp.bfloat16)
        qv2_ref[d, :, D:2 * D] = v.astype(jnp.bfloat16)
        s2_ref[d] = s


def _tc_pre_body(x_ref, w1_ref, b1_ref, w2_ref, b2m_ref, wd_ref, bd_ref,
                 b2_ref, k2_ref, qv2_ref, s2_ref):
    x = x_ref[...]
    h = jnp.maximum(_dotT(x, w1_ref[...]) + b1_ref[...], 0.0)
    h = _dotT(h, w2_ref[...]) + b2m_ref[...]
    _emit_layer([h, h], wd_ref[...], bd_ref[...], b2_ref[...],
                k2_ref, qv2_ref, s2_ref)


def _tc_mid_body(agg_ref, sp_ref, wd_ref, bd_ref, b2_ref,
                 k2_ref, qv2_ref, s2_ref):
    h2 = [agg_ref[d] + sp_ref[d] for d in range(2)]
    _emit_layer(h2, wd_ref[...], bd_ref[...], b2_ref[...],
                k2_ref, qv2_ref, s2_ref)


def _tc_fin_body(agg_ref, sp_ref, scw_ref, scb_ref, out_ref):
    hf = agg_ref[0] + sp_ref[0]
    hb = agg_ref[1] + sp_ref[1]
    scw = scw_ref[...]
    wf = scw[:, 0:D]
    wb = scw[:, D:2 * D]
    t = (lax.dot_general(wf, hf, (((1,), (1,)), ((), ())),
                         preferred_element_type=jnp.float32)
         + lax.dot_general(wb, hb, (((1,), (1,)), ((), ())),
                           preferred_element_type=jnp.float32))
    out_ref[...] = t + scb_ref[...]


def _full(shape):
    return pl.BlockSpec(shape, lambda i: (0,) * len(shape))


_GRID = NP // R

_LAYER_OUT_SHAPE = (
    jax.ShapeDtypeStruct((2, NP, D), jnp.float32),
    jax.ShapeDtypeStruct((2, NP, 2 * D), jnp.bfloat16),
    jax.ShapeDtypeStruct((2, NP, D), jnp.float32),
)
_LAYER_OUT_SPECS = [
    pl.BlockSpec((2, R, D), lambda i: (0, i, 0)),
    pl.BlockSpec((2, R, 2 * D), lambda i: (0, i, 0)),
    pl.BlockSpec((2, R, D), lambda i: (0, i, 0)),
]

_tc_pre = pl.pallas_call(
    _tc_pre_body,
    grid=(_GRID,),
    in_specs=[
        pl.BlockSpec((R, D), lambda i: (i, 0)),
        _full((D, D)), _full((1, D)), _full((D, D)), _full((1, D)),
        _full((2, 4, D, D)), _full((2, 3, 1, D)), _full((2, 1, D)),
    ],
    out_specs=_LAYER_OUT_SPECS,
    out_shape=_LAYER_OUT_SHAPE,
)

_tc_mid = pl.pallas_call(
    _tc_mid_body,
    grid=(_GRID,),
    in_specs=[
        pl.BlockSpec((2, R, D), lambda i: (0, i, 0)),
        pl.BlockSpec((2, R, D), lambda i: (0, i, 0)),
        _full((2, 4, D, D)), _full((2, 3, 1, D)), _full((2, 1, D)),
    ],
    out_specs=_LAYER_OUT_SPECS,
    out_shape=_LAYER_OUT_SHAPE,
)

_tc_fin = pl.pallas_call(
    _tc_fin_body,
    grid=(_GRID,),
    in_specs=[
        pl.BlockSpec((2, R, D), lambda i: (0, i, 0)),
        pl.BlockSpec((2, R, D), lambda i: (0, i, 0)),
        _full((1, 2 * D)), _full((1, 1)),
    ],
    out_specs=pl.BlockSpec((1, R), lambda i: (0, i)),
    out_shape=jax.ShapeDtypeStruct((1, NP), jnp.float32),
)


@functools.cache
def _make_sc_edge(nc: int):
    pairs = nc // 2
    rows_pt = AGGR // NSUB

    mesh = plsc.VectorSubcoreMesh(core_axis_name="c", subcore_axis_name="s",
                                  num_cores=2, num_subcores=NSUB)

    @functools.partial(
        pl.kernel,
        mesh=mesh,
        out_type=jax.ShapeDtypeStruct((2, NP, D), jnp.float32),
        scratch_types=[
            pltpu.VMEM((8, CH), jnp.int32),
            pltpu.VMEM((8, CH), jnp.int32),
            pltpu.VMEM((CH, D), jnp.float32),
            pltpu.VMEM((CH, D), jnp.float32),
            pltpu.VMEM((CH, D), jnp.int32),
            pltpu.VMEM((CH, D), jnp.int32),
            pltpu.VMEM((CH, D), jnp.float32),
            pltpu.VMEM_SHARED((AGGR, D), jnp.float32),
            pltpu.SemaphoreType.DMA,
            pltpu.SemaphoreType.DMA,
            pltpu.SemaphoreType.DMA,
            pltpu.SemaphoreType.DMA,
            pltpu.SemaphoreType.DMA,
            pltpu.SemaphoreType.DMA,
        ],
    )
    def sc_edge(k2, qv2, ipack, zrows, out,
                i0, i1, k0, k1, q0, q1, msg, agg,
                si0, si1, sk0, sk1, sq0, sq1):
        idxb = (i0, i1)
        krowb = (k0, k1)
        qvb = (q0, q1)
        semi = (si0, si1)
        semk = (sk0, sk1)
        semq = (sq0, sq1)
        cid = lax.axis_index("c")
        sid = lax.axis_index("s")
        pltpu.sync_copy(zrows, agg.at[pl.ds(sid * rows_pt, rows_pt)])
        plsc.subcore_barrier()

        def slab(t):
            return ipack.at[cid, sid, t]

        def i_start(b, t):
            pltpu.async_copy(slab(t), idxb[b], semi[b])

        def i_wait(b, t):
            pltpu.make_async_copy(slab(t), idxb[b], semi[b]).wait()

        def g_start(b):
            pltpu.async_copy(k2.at[idxb[b].at[0]], krowb[b], semk[b])
            pltpu.async_copy(qv2.at[idxb[b].at[1]], qvb[b], semq[b])

        def g_wait(b):
            pltpu.make_async_copy(k2.at[idxb[b].at[0]], krowb[b], semk[b]).wait()
            pltpu.make_async_copy(qv2.at[idxb[b].at[1]], qvb[b], semq[b]).wait()

        i_start(0, 0)
        i_start(1, 1)
        i_wait(0, 0)
        g_start(0)

        def pair(p, carry):
            for b in (0, 1):
                t = 2 * p + b
                if b == 0:
                    i_wait(1, t + 1)
                    g_start(1)
                else:
                    @pl.when(p < pairs - 1)
                    def _issue_next():
                        i_wait(0, t + 1)
                        g_start(0)
                g_wait(b)

                @plsc.parallel_loop(0, CH, unroll=4)
                def _rows(e):
                    hi_mask = jnp.int32(-65536)
                    for u in range(D // 32):
                        wq = qvb[b][e, pl.ds(u * 16, 16)]
                        wv = qvb[b][e, pl.ds(D // 2 + u * 16, 16)]
                        ka = krowb[b][e, pl.ds(u * 32, 16)]
                        kb_ = krowb[b][e, pl.ds(u * 32 + 16, 16)]
                        qa = lax.bitcast_convert_type(wq << 16, jnp.float32)
                        qb_ = lax.bitcast_convert_type(wq & hi_mask, jnp.float32)
                        va = lax.bitcast_convert_type(wv << 16, jnp.float32)
                        vb_ = lax.bitcast_convert_type(wv & hi_mask, jnp.float32)
                        msg[e, pl.ds(u * 32, 16)] = \
                            va / (1.0 + jnp.exp(-(ka + qa)))
                        msg[e, pl.ds(u * 32 + 16, 16)] = \
                            vb_ / (1.0 + jnp.exp(-(kb_ + qb_)))
                pltpu.sync_copy(msg, agg.at[idxb[b].at[2]], add=True)

                @pl.when(p < pairs - 1)
                def _prefetch_idx():
                    i_start(b, t + 2)
            return carry

        lax.fori_loop(0, pairs, pair, 0, unroll=False)
        plsc.subcore_barrier()
        pltpu.sync_copy(agg.at[pl.ds(sid * rows_pt, rows_pt)],
                        out.at[cid, pl.ds(sid * rows_pt, rows_pt)])

    return sc_edge


def _stack_dir(pf, pb):
    def perm(p):
        return {'Kw': p['Kw'], 'Kb': p['Kb'],
                'Qw': p['Qw'][_ILV], 'Qb': p['Qb'][_ILV],
                'Vw': p['Vw'][_ILV], 'Vb': p['Vb'][_ILV],
                'Sw': p['Sw'], 'bias': p['bias']}
    pf, pb = perm(pf), perm(pb)
    wd = jnp.stack([
        jnp.stack([pf['Kw'], pf['Qw'], pf['Vw'], pf['Sw']]),
        jnp.stack([pb['Kw'], pb['Qw'], pb['Vw'], pb['Sw']]),
    ])
    bd = jnp.stack([
        jnp.stack([pf['Kb'], pf['Qb'], pf['Vb']]),
        jnp.stack([pb['Kb'], pb['Qb'], pb['Vb']]),
    ]).reshape(2, 3, 1, D)
    b2 = jnp.stack([pf['bias'], pb['bias']]).reshape(2, 1, D)
    return wd, bd, b2


def kernel(x, edge_index, params):
    xp = jnp.pad(x, ((0, NP - N), (0, 0)))
    src, dst = edge_index[0], edge_index[1]
    e = src.shape[0]
    quantum = NSUB * CH * 2
    ep = -(-e // quantum) * quantum
    nc = ep // (NSUB * CH)
    pad = ep - e
    src_p = jnp.pad(src, (0, pad), constant_values=TRASH)
    dst_p = jnp.pad(dst, (0, pad), constant_values=TRASH)
    gk = jnp.stack([dst_p, src_p + NP]).reshape(2, NSUB, nc, CH)
    gqv = jnp.stack([src_p, dst_p + NP]).reshape(2, NSUB, nc, CH)
    gsc = jnp.stack([dst_p, src_p]).reshape(2, NSUB, nc, CH)
    ipack = jnp.concatenate(
        [jnp.stack([gk, gqv, gsc], axis=3),
         jnp.zeros((2, NSUB, nc, 5, CH), jnp.int32)], axis=3)
    zrows = jnp.zeros((AGGR // NSUB, D), jnp.float32)

    w1 = params['W1w']
    b1 = params['W1b'].reshape(1, D)
    w2 = params['W2w']
    b2m = params['W2b'].reshape(1, D)
    layers = [_stack_dir(params['fw'][l], params['bw'][l]) for l in range(3)]
    scw = params['Scw']
    scb = params['Scb'].reshape(1, 1)

    sc_edge = _make_sc_edge(nc)

    def as_i32(a, w):
        return lax.bitcast_convert_type(
            a.reshape(2 * NP, w // 2, 2), jnp.int32)

    wd, bd, b2 = layers[0]
    k2, qv2, s2 = _tc_pre(xp, w1, b1, w2, b2m, wd, bd, b2)
    for l in (1, 2):
        agg = sc_edge(k2.reshape(2 * NP, D), as_i32(qv2, 2 * D), ipack, zrows)
        wd, bd, b2 = layers[l]
        k2, qv2, s2 = _tc_mid(agg, s2, wd, bd, b2)
    agg = sc_edge(k2.reshape(2 * NP, D), as_i32(qv2, 2 * D), ipack, zrows)
    score = _tc_fin(agg, s2, scw, scb)
    return score.reshape(NP, 1)[:N]


# speedup vs baseline: 6.1402x; 1.0549x over previous
import functools

import jax
import jax.numpy as jnp
import numpy as np
from jax import lax
from jax.experimental import pallas as pl
from jax.experimental.pallas import tpu as pltpu
from jax.experimental.pallas import tpu_sc as plsc

N = 10000
NP = 10240
D = 128
R = 1024
CH = 72
AGGR = 10112
NSUB = 16
TRASH = N

_ILV = np.empty((D,), np.int32)
for _u in range(D // 32):
    for _i in range(16):
        _ILV[32 * _u + 2 * _i] = 32 * _u + _i
        _ILV[32 * _u + 2 * _i + 1] = 32 * _u + 16 + _i


def _dotT(a, w):
    return lax.dot_general(a, w, (((1,), (1,)), ((), ())),
                           preferred_element_type=jnp.float32)


def _emit_layer(h2, wd, bd, b2, k2_ref, qv2_ref, s2_ref):
    for d in range(2):
        h = h2[d]
        k = _dotT(h, wd[d, 0]) + bd[d, 0]
        q = _dotT(h, wd[d, 1]) + bd[d, 1]
        v = _dotT(h, wd[d, 2]) + bd[d, 2]
        s = _dotT(h, wd[d, 3]) + b2[d]
        k2_ref[d] = k
        qv2_ref[d, :, 0:D] = q.astype(jnp.bfloat16)
        qv2_ref[d, :, D:2 * D] = v.astype(jnp.bfloat16)
        s2_ref[d] = s


def _tc_pre_body(x_ref, w1_ref, b1_ref, w2_ref, b2m_ref, wd_ref, bd_ref,
                 b2_ref, k2_ref, qv2_ref, s2_ref):
    x = x_ref[...]
    h = jnp.maximum(_dotT(x, w1_ref[...]) + b1_ref[...], 0.0)
    h = _dotT(h, w2_ref[...]) + b2m_ref[...]
    _emit_layer([h, h], wd_ref[...], bd_ref[...], b2_ref[...],
                k2_ref, qv2_ref, s2_ref)


def _tc_mid_body(agg_ref, sp_ref, wd_ref, bd_ref, b2_ref,
                 k2_ref, qv2_ref, s2_ref):
    h2 = [agg_ref[d] + sp_ref[d] for d in range(2)]
    _emit_layer(h2, wd_ref[...], bd_ref[...], b2_ref[...],
                k2_ref, qv2_ref, s2_ref)


def _tc_fin_body(agg_ref, sp_ref, scw_ref, scb_ref, out_ref):
    hf = agg_ref[0] + sp_ref[0]
    hb = agg_ref[1] + sp_ref[1]
    scw = scw_ref[...]
    wf = scw[:, 0:D]
    wb = scw[:, D:2 * D]
    t = (lax.dot_general(wf, hf, (((1,), (1,)), ((), ())),
                         preferred_element_type=jnp.float32)
         + lax.dot_general(wb, hb, (((1,), (1,)), ((), ())),
                           preferred_element_type=jnp.float32))
    out_ref[...] = t + scb_ref[...]


def _full(shape):
    return pl.BlockSpec(shape, lambda i: (0,) * len(shape))


_GRID = NP // R

_LAYER_OUT_SHAPE = (
    jax.ShapeDtypeStruct((2, NP, D), jnp.float32),
    jax.ShapeDtypeStruct((2, NP, 2 * D), jnp.bfloat16),
    jax.ShapeDtypeStruct((2, NP, D), jnp.float32),
)
_LAYER_OUT_SPECS = [
    pl.BlockSpec((2, R, D), lambda i: (0, i, 0)),
    pl.BlockSpec((2, R, 2 * D), lambda i: (0, i, 0)),
    pl.BlockSpec((2, R, D), lambda i: (0, i, 0)),
]

_tc_pre = pl.pallas_call(
    _tc_pre_body,
    grid=(_GRID,),
    in_specs=[
        pl.BlockSpec((R, D), lambda i: (i, 0)),
        _full((D, D)), _full((1, D)), _full((D, D)), _full((1, D)),
        _full((2, 4, D, D)), _full((2, 3, 1, D)), _full((2, 1, D)),
    ],
    out_specs=_LAYER_OUT_SPECS,
    out_shape=_LAYER_OUT_SHAPE,
)

_tc_mid = pl.pallas_call(
    _tc_mid_body,
    grid=(_GRID,),
    in_specs=[
        pl.BlockSpec((2, R, D), lambda i: (0, i, 0)),
        pl.BlockSpec((2, R, D), lambda i: (0, i, 0)),
        _full((2, 4, D, D)), _full((2, 3, 1, D)), _full((2, 1, D)),
    ],
    out_specs=_LAYER_OUT_SPECS,
    out_shape=_LAYER_OUT_SHAPE,
)

_tc_fin = pl.pallas_call(
    _tc_fin_body,
    grid=(_GRID,),
    in_specs=[
        pl.BlockSpec((2, R, D), lambda i: (0, i, 0)),
        pl.BlockSpec((2, R, D), lambda i: (0, i, 0)),
        _full((1, 2 * D)), _full((1, 1)),
    ],
    out_specs=pl.BlockSpec((1, R), lambda i: (0, i)),
    out_shape=jax.ShapeDtypeStruct((1, NP), jnp.float32),
)


@functools.cache
def _make_sc_edge(nc: int):
    pairs = nc // 2
    rows_pt = AGGR // NSUB

    mesh = plsc.VectorSubcoreMesh(core_axis_name="c", subcore_axis_name="s",
                                  num_cores=2, num_subcores=NSUB)

    @functools.partial(
        pl.kernel,
        mesh=mesh,
        out_type=jax.ShapeDtypeStruct((2, NP, D), jnp.float32),
        scratch_types=[
            pltpu.VMEM((8, CH), jnp.int32),
            pltpu.VMEM((8, CH), jnp.int32),
            pltpu.VMEM((CH, D), jnp.float32),
            pltpu.VMEM((CH, D), jnp.float32),
            pltpu.VMEM((CH, D), jnp.int32),
            pltpu.VMEM((CH, D), jnp.int32),
            pltpu.VMEM((CH, D), jnp.float32),
            pltpu.VMEM_SHARED((AGGR, D), jnp.float32),
            pltpu.SemaphoreType.DMA,
            pltpu.SemaphoreType.DMA,
            pltpu.SemaphoreType.DMA,
            pltpu.SemaphoreType.DMA,
            pltpu.SemaphoreType.DMA,
            pltpu.SemaphoreType.DMA,
        ],
    )
    def sc_edge(k2, qv2, ipack, zrows, out,
                i0, i1, k0, k1, q0, q1, msg, agg,
                si0, si1, sk0, sk1, sq0, sq1):
        idxb = (i0, i1)
        krowb = (k0, k1)
        qvb = (q0, q1)
        semi = (si0, si1)
        semk = (sk0, sk1)
        semq = (sq0, sq1)
        cid = lax.axis_index("c")
        sid = lax.axis_index("s")
        pltpu.sync_copy(zrows, agg.at[pl.ds(sid * rows_pt, rows_pt)])
        plsc.subcore_barrier()

        def slab(t):
            return ipack.at[cid, sid, t]

        def i_start(b, t):
            pltpu.async_copy(slab(t), idxb[b], semi[b])

        def i_wait(b, t):
            pltpu.make_async_copy(slab(t), idxb[b], semi[b]).wait()

        def g_start(b):
            pltpu.async_copy(k2.at[idxb[b].at[0]], krowb[b], semk[b])
            pltpu.async_copy(qv2.at[idxb[b].at[1]], qvb[b], semq[b])

        def g_wait(b):
            pltpu.make_async_copy(k2.at[idxb[b].at[0]], krowb[b], semk[b]).wait()
            pltpu.make_async_copy(qv2.at[idxb[b].at[1]], qvb[b], semq[b]).wait()

        i_start(0, 0)
        i_start(1, 1)
        i_wait(0, 0)
        g_start(0)

        def pair(p, carry):
            for b in (0, 1):
                t = 2 * p + b
                if b == 0:
                    i_wait(1, t + 1)
                    g_start(1)
                else:
                    @pl.when(p < pairs - 1)
                    def _issue_next():
                        i_wait(0, t + 1)
                        g_start(0)
                g_wait(b)

                @plsc.parallel_loop(0, CH, unroll=4)
                def _rows(e):
                    hi_mask = jnp.int32(-65536)
                    for u in range(D // 32):
                        wq = qvb[b][e, pl.ds(u * 16, 16)]
                        wv = qvb[b][e, pl.ds(D // 2 + u * 16, 16)]
                        ka = krowb[b][e, pl.ds(u * 32, 16)]
                        kb_ = krowb[b][e, pl.ds(u * 32 + 16, 16)]
                        qa = lax.bitcast_convert_type(wq << 16, jnp.float32)
                        qb_ = lax.bitcast_convert_type(wq & hi_mask, jnp.float32)
                        va = lax.bitcast_convert_type(wv << 16, jnp.float32)
                        vb_ = lax.bitcast_convert_type(wv & hi_mask, jnp.float32)
                        msg[e, pl.ds(u * 32, 16)] = \
                            va / (1.0 + jnp.exp(-(ka + qa)))
                        msg[e, pl.ds(u * 32 + 16, 16)] = \
                            vb_ / (1.0 + jnp.exp(-(kb_ + qb_)))
                pltpu.sync_copy(msg, agg.at[idxb[b].at[2]], add=True)

                @pl.when(p < pairs - 1)
                def _prefetch_idx():
                    i_start(b, t + 2)
            return carry

        lax.fori_loop(0, pairs, pair, 0, unroll=False)
        plsc.subcore_barrier()
        pltpu.sync_copy(agg.at[pl.ds(sid * rows_pt, rows_pt)],
                        out.at[cid, pl.ds(sid * rows_pt, rows_pt)])

    return sc_edge


def _stack_dir(pf, pb):
    def perm(p):
        return {'Kw': p['Kw'], 'Kb': p['Kb'],
                'Qw': p['Qw'][_ILV], 'Qb': p['Qb'][_ILV],
                'Vw': p['Vw'][_ILV], 'Vb': p['Vb'][_ILV],
                'Sw': p['Sw'], 'bias': p['bias']}
    pf, pb = perm(pf), perm(pb)
    wd = jnp.stack([
        jnp.stack([pf['Kw'], pf['Qw'], pf['Vw'], pf['Sw']]),
        jnp.stack([pb['Kw'], pb['Qw'], pb['Vw'], pb['Sw']]),
    ])
    bd = jnp.stack([
        jnp.stack([pf['Kb'], pf['Qb'], pf['Vb']]),
        jnp.stack([pb['Kb'], pb['Qb'], pb['Vb']]),
    ]).reshape(2, 3, 1, D)
    b2 = jnp.stack([pf['bias'], pb['bias']]).reshape(2, 1, D)
    return wd, bd, b2


def kernel(x, edge_index, params):
    xp = jnp.pad(x, ((0, NP - N), (0, 0)))
    src, dst = edge_index[0], edge_index[1]
    e = src.shape[0]
    quantum = NSUB * CH * 2
    ep = -(-e // quantum) * quantum
    nc = ep // (NSUB * CH)
    pad = ep - e
    src_p = jnp.pad(src, (0, pad), constant_values=TRASH)
    dst_p = jnp.pad(dst, (0, pad), constant_values=TRASH)
    gk = jnp.stack([dst_p, src_p + NP]).reshape(2, NSUB, nc, CH)
    gqv = jnp.stack([src_p, dst_p + NP]).reshape(2, NSUB, nc, CH)
    gsc = jnp.stack([dst_p, src_p]).reshape(2, NSUB, nc, CH)
    ipack = jnp.concatenate(
        [jnp.stack([gk, gqv, gsc], axis=3),
         jnp.zeros((2, NSUB, nc, 5, CH), jnp.int32)], axis=3)
    zrows = jnp.zeros((AGGR // NSUB, D), jnp.float32)

    w1 = params['W1w']
    b1 = params['W1b'].reshape(1, D)
    w2 = params['W2w']
    b2m = params['W2b'].reshape(1, D)
    layers = [_stack_dir(params['fw'][l], params['bw'][l]) for l in range(3)]
    scw = params['Scw']
    scb = params['Scb'].reshape(1, 1)

    sc_edge = _make_sc_edge(nc)

    def as_i32(a, w):
        return lax.bitcast_convert_type(
            a.reshape(2 * NP, w // 2, 2), jnp.int32)

    wd, bd, b2 = layers[0]
    k2, qv2, s2 = _tc_pre(xp, w1, b1, w2, b2m, wd, bd, b2)
    for l in (1, 2):
        agg = sc_edge(k2.reshape(2 * NP, D), as_i32(qv2, 2 * D), ipack, zrows)
        wd, bd, b2 = layers[l]
        k2, qv2, s2 = _tc_mid(agg, s2, wd, bd, b2)
    agg = sc_edge(k2.reshape(2 * NP, D), as_i32(qv2, 2 * D), ipack, zrows)
    score = _tc_fin(agg, s2, scw, scb)
    return score.reshape(NP, 1)[:N]


# speedup vs baseline: 6.2268x; 1.0141x over previous
import functools

import jax
import jax.numpy as jnp
import numpy as np
from jax import lax
from jax.experimental import pallas as pl
from jax.experimental.pallas import tpu as pltpu
from jax.experimental.pallas import tpu_sc as plsc

N = 10000
NP = 10240
D = 128
R = 1024
CH = 72
AGGR = 10112
NSUB = 16
TRASH = N

_ILV = np.empty((D,), np.int32)
for _u in range(D // 32):
    for _i in range(16):
        _ILV[32 * _u + 2 * _i] = 32 * _u + _i
        _ILV[32 * _u + 2 * _i + 1] = 32 * _u + 16 + _i


def _dotT(a, w):
    return lax.dot_general(a, w, (((1,), (1,)), ((), ())),
                           preferred_element_type=jnp.float32)


def _emit_layer(h2, wd, bd, b2, k2_ref, qv2_ref, s2_ref):
    for d in range(2):
        h = h2[d]
        k = _dotT(h, wd[d, 0]) + bd[d, 0]
        q = _dotT(h, wd[d, 1]) + bd[d, 1]
        v = _dotT(h, wd[d, 2]) + bd[d, 2]
        s = _dotT(h, wd[d, 3]) + b2[d]
        k2_ref[d] = k
        qv2_ref[d, :, 0:D] = q.astype(jnp.bfloat16)
        qv2_ref[d, :, D:2 * D] = v.astype(jnp.bfloat16)
        s2_ref[d] = s


def _tc_pre_body(x_ref, w1_ref, b1_ref, w2_ref, b2m_ref, wd_ref, bd_ref,
                 b2_ref, k2_ref, qv2_ref, s2_ref):
    x = x_ref[...]
    h = jnp.maximum(_dotT(x, w1_ref[...]) + b1_ref[...], 0.0)
    h = _dotT(h, w2_ref[...]) + b2m_ref[...]
    _emit_layer([h, h], wd_ref[...], bd_ref[...], b2_ref[...],
                k2_ref, qv2_ref, s2_ref)


def _tc_mid_body(agg_ref, sp_ref, wd_ref, bd_ref, b2_ref,
                 k2_ref, qv2_ref, s2_ref):
    h2 = [agg_ref[d] + sp_ref[d] for d in range(2)]
    _emit_layer(h2, wd_ref[...], bd_ref[...], b2_ref[...],
                k2_ref, qv2_ref, s2_ref)


def _tc_fin_body(agg_ref, sp_ref, scw_ref, scb_ref, out_ref):
    hf = agg_ref[0] + sp_ref[0]
    hb = agg_ref[1] + sp_ref[1]
    scw = scw_ref[...]
    wf = scw[:, 0:D]
    wb = scw[:, D:2 * D]
    t = (lax.dot_general(wf, hf, (((1,), (1,)), ((), ())),
                         preferred_element_type=jnp.float32)
         + lax.dot_general(wb, hb, (((1,), (1,)), ((), ())),
                           preferred_element_type=jnp.float32))
    out_ref[...] = t + scb_ref[...]


def _full(shape):
    return pl.BlockSpec(shape, lambda i: (0,) * len(shape))


_GRID = NP // R

_LAYER_OUT_SHAPE = (
    jax.ShapeDtypeStruct((2, NP, D), jnp.float32),
    jax.ShapeDtypeStruct((2, NP, 2 * D), jnp.bfloat16),
    jax.ShapeDtypeStruct((2, NP, D), jnp.float32),
)
_LAYER_OUT_SPECS = [
    pl.BlockSpec((2, R, D), lambda i: (0, i, 0)),
    pl.BlockSpec((2, R, 2 * D), lambda i: (0, i, 0)),
    pl.BlockSpec((2, R, D), lambda i: (0, i, 0)),
]

_tc_pre = pl.pallas_call(
    _tc_pre_body,
    grid=(_GRID,),
    in_specs=[
        pl.BlockSpec((R, D), lambda i: (i, 0)),
        _full((D, D)), _full((1, D)), _full((D, D)), _full((1, D)),
        _full((2, 4, D, D)), _full((2, 3, 1, D)), _full((2, 1, D)),
    ],
    out_specs=_LAYER_OUT_SPECS,
    out_shape=_LAYER_OUT_SHAPE,
)

_tc_mid = pl.pallas_call(
    _tc_mid_body,
    grid=(_GRID,),
    in_specs=[
        pl.BlockSpec((2, R, D), lambda i: (0, i, 0)),
        pl.BlockSpec((2, R, D), lambda i: (0, i, 0)),
        _full((2, 4, D, D)), _full((2, 3, 1, D)), _full((2, 1, D)),
    ],
    out_specs=_LAYER_OUT_SPECS,
    out_shape=_LAYER_OUT_SHAPE,
)

_tc_fin = pl.pallas_call(
    _tc_fin_body,
    grid=(_GRID,),
    in_specs=[
        pl.BlockSpec((2, R, D), lambda i: (0, i, 0)),
        pl.BlockSpec((2, R, D), lambda i: (0, i, 0)),
        _full((1, 2 * D)), _full((1, 1)),
    ],
    out_specs=pl.BlockSpec((1, R), lambda i: (0, i)),
    out_shape=jax.ShapeDtypeStruct((1, NP), jnp.float32),
)


@functools.cache
def _make_sc_edge(nc: int):
    pairs = nc // 2
    rows_pt = AGGR // NSUB

    mesh = plsc.VectorSubcoreMesh(core_axis_name="c", subcore_axis_name="s",
                                  num_cores=2, num_subcores=NSUB)

    @functools.partial(
        pl.kernel,
        mesh=mesh,
        out_type=jax.ShapeDtypeStruct((2, NP, D), jnp.float32),
        scratch_types=[
            pltpu.VMEM((8, CH), jnp.int32),
            pltpu.VMEM((8, CH), jnp.int32),
            pltpu.VMEM((CH, D), jnp.float32),
            pltpu.VMEM((CH, D), jnp.float32),
            pltpu.VMEM((CH, D), jnp.int32),
            pltpu.VMEM((CH, D), jnp.int32),
            pltpu.VMEM((CH, D), jnp.float32),
            pltpu.VMEM_SHARED((AGGR, D), jnp.float32),
            pltpu.SemaphoreType.DMA,
            pltpu.SemaphoreType.DMA,
            pltpu.SemaphoreType.DMA,
            pltpu.SemaphoreType.DMA,
            pltpu.SemaphoreType.DMA,
            pltpu.SemaphoreType.DMA,
        ],
    )
    def sc_edge(k2, qv2, ipack, zrows, out,
                i0, i1, k0, k1, q0, q1, msg, agg,
                si0, si1, sk0, sk1, sq0, sq1):
        idxb = (i0, i1)
        krowb = (k0, k1)
        qvb = (q0, q1)
        semi = (si0, si1)
        semk = (sk0, sk1)
        semq = (sq0, sq1)
        cid = lax.axis_index("c")
        sid = lax.axis_index("s")
        pltpu.sync_copy(zrows, agg.at[pl.ds(sid * rows_pt, rows_pt)])
        plsc.subcore_barrier()

        def slab(t):
            return ipack.at[cid, sid, t]

        def i_start(b, t):
            pltpu.async_copy(slab(t), idxb[b], semi[b])

        def i_wait(b, t):
            pltpu.make_async_copy(slab(t), idxb[b], semi[b]).wait()

        def g_start(b):
            pltpu.async_copy(k2.at[idxb[b].at[0]], krowb[b], semk[b])
            pltpu.async_copy(qv2.at[idxb[b].at[1]], qvb[b], semq[b])

        def g_wait(b):
            pltpu.make_async_copy(k2.at[idxb[b].at[0]], krowb[b], semk[b]).wait()
            pltpu.make_async_copy(qv2.at[idxb[b].at[1]], qvb[b], semq[b]).wait()

        i_start(0, 0)
        i_start(1, 1)
        i_wait(0, 0)
        g_start(0)

        def pair(p, carry):
            for b in (0, 1):
                t = 2 * p + b
                if b == 0:
                    i_wait(1, t + 1)
                    g_start(1)
                else:
                    @pl.when(p < pairs - 1)
                    def _issue_next():
                        i_wait(0, t + 1)
                        g_start(0)
                g_wait(b)

                @plsc.parallel_loop(0, CH, unroll=6)
                def _rows(e):
                    hi_mask = jnp.int32(-65536)
                    for u in range(D // 32):
                        wq = qvb[b][e, pl.ds(u * 16, 16)]
                        wv = qvb[b][e, pl.ds(D // 2 + u * 16, 16)]
                        ka = krowb[b][e, pl.ds(u * 32, 16)]
                        kb_ = krowb[b][e, pl.ds(u * 32 + 16, 16)]
                        qa = lax.bitcast_convert_type(wq << 16, jnp.float32)
                        qb_ = lax.bitcast_convert_type(wq & hi_mask, jnp.float32)
                        va = lax.bitcast_convert_type(wv << 16, jnp.float32)
                        vb_ = lax.bitcast_convert_type(wv & hi_mask, jnp.float32)
                        msg[e, pl.ds(u * 32, 16)] = \
                            va / (1.0 + jnp.exp(-(ka + qa)))
                        msg[e, pl.ds(u * 32 + 16, 16)] = \
                            vb_ / (1.0 + jnp.exp(-(kb_ + qb_)))
                pltpu.sync_copy(msg, agg.at[idxb[b].at[2]], add=True)

                @pl.when(p < pairs - 1)
                def _prefetch_idx():
                    i_start(b, t + 2)
            return carry

        lax.fori_loop(0, pairs, pair, 0, unroll=False)
        plsc.subcore_barrier()
        pltpu.sync_copy(agg.at[pl.ds(sid * rows_pt, rows_pt)],
                        out.at[cid, pl.ds(sid * rows_pt, rows_pt)])

    return sc_edge


def _stack_dir(pf, pb):
    def perm(p):
        return {'Kw': p['Kw'], 'Kb': p['Kb'],
                'Qw': p['Qw'][_ILV], 'Qb': p['Qb'][_ILV],
                'Vw': p['Vw'][_ILV], 'Vb': p['Vb'][_ILV],
                'Sw': p['Sw'], 'bias': p['bias']}
    pf, pb = perm(pf), perm(pb)
    wd = jnp.stack([
        jnp.stack([pf['Kw'], pf['Qw'], pf['Vw'], pf['Sw']]),
        jnp.stack([pb['Kw'], pb['Qw'], pb['Vw'], pb['Sw']]),
    ])
    bd = jnp.stack([
        jnp.stack([pf['Kb'], pf['Qb'], pf['Vb']]),
        jnp.stack([pb['Kb'], pb['Qb'], pb['Vb']]),
    ]).reshape(2, 3, 1, D)
    b2 = jnp.stack([pf['bias'], pb['bias']]).reshape(2, 1, D)
    return wd, bd, b2


def kernel(x, edge_index, params):
    xp = jnp.pad(x, ((0, NP - N), (0, 0)))
    src, dst = edge_index[0], edge_index[1]
    e = src.shape[0]
    quantum = NSUB * CH * 2
    ep = -(-e // quantum) * quantum
    nc = ep // (NSUB * CH)
    pad = ep - e
    src_p = jnp.pad(src, (0, pad), constant_values=TRASH)
    dst_p = jnp.pad(dst, (0, pad), constant_values=TRASH)
    gk = jnp.stack([dst_p, src_p + NP]).reshape(2, NSUB, nc, CH)
    gqv = jnp.stack([src_p, dst_p + NP]).reshape(2, NSUB, nc, CH)
    gsc = jnp.stack([dst_p, src_p]).reshape(2, NSUB, nc, CH)
    ipack = jnp.concatenate(
        [jnp.stack([gk, gqv, gsc], axis=3),
         jnp.zeros((2, NSUB, nc, 5, CH), jnp.int32)], axis=3)
    zrows = jnp.zeros((AGGR // NSUB, D), jnp.float32)

    w1 = params['W1w']
    b1 = params['W1b'].reshape(1, D)
    w2 = params['W2w']
    b2m = params['W2b'].reshape(1, D)
    layers = [_stack_dir(params['fw'][l], params['bw'][l]) for l in range(3)]
    scw = params['Scw']
    scb = params['Scb'].reshape(1, 1)

    sc_edge = _make_sc_edge(nc)

    def as_i32(a, w):
        return lax.bitcast_convert_type(
            a.reshape(2 * NP, w // 2, 2), jnp.int32)

    wd, bd, b2 = layers[0]
    k2, qv2, s2 = _tc_pre(xp, w1, b1, w2, b2m, wd, bd, b2)
    for l in (1, 2):
        agg = sc_edge(k2.reshape(2 * NP, D), as_i32(qv2, 2 * D), ipack, zrows)
        wd, bd, b2 = layers[l]
        k2, qv2, s2 = _tc_mid(agg, s2, wd, bd, b2)
    agg = sc_edge(k2.reshape(2 * NP, D), as_i32(qv2, 2 * D), ipack, zrows)
    score = _tc_fin(agg, s2, scw, scb)
    return score.reshape(NP, 1)[:N]
